# Initial kernel scaffold; baseline (speedup 1.0000x reference)
import jax
import jax.numpy as jnp
from jax.experimental import pallas as pl


def kernel(x, edge_index, W, b):
    raise NotImplementedError("write your pallas kernel here")


# speedup vs baseline: 16.2596x; 16.2596x over previous
import functools

import jax
import jax.numpy as jnp
from jax import lax
from jax.experimental import pallas as pl
from jax.experimental.pallas import tpu as pltpu
from jax.experimental.pallas import tpu_sc as plsc

N = 10000
E = 320000
C = 128
NC = 2
NS = 16
NW = NC * NS
EPW = E // NW
CHUNK = 128
EPWP = 10240
NCHUNK = EPWP // CHUNK
HALF = NCHUNK // 2
NP = 10240
GP = 10008

_MESH = dict(core_axis_name="c", subcore_axis_name="s", num_cores=NC,
             num_subcores=NS)
_SC_PARAMS = pltpu.CompilerParams(needs_layout_passes=False)


def _deg_body(dst_hbm, hist_hbm, dst_v, hist_v):
    c = lax.axis_index("c")
    s = lax.axis_index("s")
    w = c * NS + s

    zeros = jnp.zeros((16,), jnp.float32)

    def zero_body(i, _):
        hist_v[pl.ds(i * 16, 16)] = zeros
        return 0

    lax.fori_loop(0, N // 16, zero_body, 0)

    pltpu.sync_copy(dst_hbm.at[pl.ds(w * EPW, EPW)], dst_v)

    ones = jnp.ones((16,), jnp.float32)

    def body(i, _):
        idx = dst_v[pl.ds(i * 16, 16)]
        plsc.addupdate_scatter(hist_v, [idx], ones)
        return 0

    lax.fori_loop(0, EPW // 16, body, 0)
    pltpu.sync_copy(hist_v, hist_hbm.at[pl.ds(w * N, N)])


def _degree_hist(dst_flat):
    return pl.kernel(
        _deg_body,
        out_type=jax.ShapeDtypeStruct((NW * N,), jnp.float32),
        mesh=plsc.VectorSubcoreMesh(**_MESH),
        compiler_params=_SC_PARAMS,
        scratch_types=[
            pltpu.VMEM((EPW,), jnp.int32),
            pltpu.VMEM((N,), jnp.float32),
        ],
    )(dst_flat)


def _lin_body(x_ref, w_ref, hist_ref, g_ref):
    deg = 1.0 + jnp.sum(hist_ref[...], axis=1)
    dis = lax.rsqrt(deg)
    h = lax.dot_general(x_ref[...], w_ref[...], (((1,), (1,)), ((), ())),
                        preferred_element_type=jnp.float32)
    g_ref[...] = h * dis[:, None]


def _linear_scaled(x, W, hist_t):
    blk = 1000
    return pl.pallas_call(
        _lin_body,
        grid=(N // blk,),
        in_specs=[
            pl.BlockSpec((blk, C), lambda i: (i, 0)),
            pl.BlockSpec((C, C), lambda i: (0, 0)),
            pl.BlockSpec((blk, NW), lambda i: (i, 0)),
        ],
        out_specs=pl.BlockSpec((blk, C), lambda i: (i, 0)),
        out_shape=jax.ShapeDtypeStruct((N, C), jnp.float32),
    )(x, W, hist_t)


def _msg_body(src_hbm, dst_hbm, g_hbm, acc_hbm, src_v, dst_v, rows0, rows1,
              sem0, sem1, acc_sh):
    c = lax.axis_index("c")
    s = lax.axis_index("s")
    w = c * NS + s

    zeros = jnp.zeros((16,), jnp.float32)

    def zero_body(i, _):
        rows0[i // 8, pl.ds((i % 8) * 16, 16)] = zeros
        return 0

    lax.fori_loop(0, CHUNK * (C // 16), zero_body, 0)
    for r in range(NP // CHUNK // NS):
        q = s + r * NS
        pltpu.sync_copy(rows0, acc_sh.at[pl.ds(q * CHUNK, CHUNK)])
    plsc.subcore_barrier()

    for h in range(NCHUNK // HALF):
        pltpu.sync_copy(src_hbm.at[w, pl.ds(h * HALF, HALF)], src_v)
        pltpu.sync_copy(dst_hbm.at[w, pl.ds(h * HALF, HALF)], dst_v)
        pltpu.async_copy(g_hbm.at[src_v.at[0]], rows0, sem0)

        def body(j, _):
            k0 = j * 2
            k1 = k0 + 1
            pltpu.async_copy(g_hbm.at[src_v.at[k1]], rows1, sem1)
            pltpu.make_async_copy(g_hbm.at[src_v.at[k0]], rows0, sem0).wait()
            pltpu.sync_copy(rows0, acc_sh.at[dst_v.at[k0]], add=True)

            @pl.when(k0 + 2 < HALF)
            def _():
                pltpu.async_copy(g_hbm.at[src_v.at[k0 + 2]], rows0, sem0)

            pltpu.make_async_copy(g_hbm.at[src_v.at[k1]], rows1, sem1).wait()
            pltpu.sync_copy(rows1, acc_sh.at[dst_v.at[k1]], add=True)
            return 0

        lax.fori_loop(0, HALF // 2, body, 0)

    plsc.subcore_barrier()
    for r in range(NP // CHUNK // NS):
        q = s + r * NS
        pltpu.sync_copy(acc_sh.at[pl.ds(q * CHUNK, CHUNK)],
                        acc_hbm.at[pl.ds(c * NP + q * CHUNK, CHUNK)])


def _edge_accumulate(src3, dst3, g_pad):
    return pl.kernel(
        _msg_body,
        out_type=jax.ShapeDtypeStruct((NC * NP, C), jnp.float32),
        mesh=plsc.VectorSubcoreMesh(**_MESH),
        compiler_params=_SC_PARAMS,
        scratch_types=[
            pltpu.VMEM((HALF, CHUNK), jnp.int32),
            pltpu.VMEM((HALF, CHUNK), jnp.int32),
            pltpu.VMEM((CHUNK, C), jnp.float32),
            pltpu.VMEM((CHUNK, C), jnp.float32),
            pltpu.SemaphoreType.DMA,
            pltpu.SemaphoreType.DMA,
            pltpu.VMEM_SHARED((NP, C), jnp.float32),
        ],
    )(src3, dst3, g_pad)


def _fin_body(acc_ref, g_ref, hist_ref, b_ref, o_ref):
    deg = 1.0 + jnp.sum(hist_ref[...], axis=1)
    dis = lax.rsqrt(deg)
    t = acc_ref[0] + acc_ref[1] + g_ref[...]
    o_ref[...] = jnp.maximum(t * dis[:, None] + b_ref[...], 0.0)


def _finalize(acc, g, hist_t, b):
    blk = 1000
    return pl.pallas_call(
        _fin_body,
        grid=(N // blk,),
        in_specs=[
            pl.BlockSpec((NC, blk, C), lambda i: (0, i, 0)),
            pl.BlockSpec((blk, C), lambda i: (i, 0)),
            pl.BlockSpec((blk, NW), lambda i: (i, 0)),
            pl.BlockSpec((1, C), lambda i: (0, 0)),
        ],
        out_specs=pl.BlockSpec((blk, C), lambda i: (i, 0)),
        out_shape=jax.ShapeDtypeStruct((N, C), jnp.float32),
    )(acc, g, hist_t, b)


@jax.jit
def kernel(x, edge_index, W, b):
    src = edge_index[0]
    dst = edge_index[1]
    hist_t = _degree_hist(dst).reshape(NW, N).T
    g = _linear_scaled(x, W, hist_t)
    src_p = jnp.pad(src.reshape(NW, EPW), ((0, 0), (0, EPWP - EPW)),
                    constant_values=N)
    dst_p = jnp.pad(dst.reshape(NW, EPW), ((0, 0), (0, EPWP - EPW)),
                    constant_values=N)
    g_pad = jnp.concatenate([g, jnp.zeros((GP - N, C), jnp.float32)], axis=0)
    acc = _edge_accumulate(src_p.reshape(NW, NCHUNK, CHUNK),
                           dst_p.reshape(NW, NCHUNK, CHUNK), g_pad)
    acc = acc.reshape(NC, NP, C)[:, :N]
    return _finalize(acc, g, hist_t, b.reshape(1, C))
